# baseline (speedup 1.0000x reference)
import jax
import jax.numpy as jnp
from jax.experimental import pallas as pl


def _gat_layer(x, edge_index, W, a, num_nodes):
    z = x @ W
    src = edge_index[0]
    dst = edge_index[1]
    z_src = z[src]
    z_dst = z[dst]
    e = jnp.concatenate([z_src, z_dst], axis=1) @ a
    e = jax.nn.leaky_relu(e[:, 0], negative_slope=0.2)
    m = jax.ops.segment_max(e, dst, num_segments=num_nodes)
    m = jnp.where(jnp.isfinite(m), m, 0.0)
    ex = jnp.exp(e - m[dst])
    denom = jax.ops.segment_sum(ex, dst, num_segments=num_nodes)
    alpha = ex / (denom[dst] + 1e-9)
    out = jax.ops.segment_sum(alpha[:, None] * z_src, dst, num_segments=num_nodes)
    return out


def kernel(edge_index, x, W1, a1, W2, a2):
    num_nodes = x.shape[0]
    h = jax.nn.relu(_gat_layer(x, edge_index, W1, a1, num_nodes))
    out = jax.nn.sigmoid(_gat_layer(h, edge_index, W2, a2, num_nodes))
    return out


# speedup vs baseline: 28.1008x; 28.1008x over previous
import jax
import jax.numpy as jnp
from jax import lax
from jax.experimental import pallas as pl
from jax.experimental.pallas import tpu as pltpu
from jax.experimental.pallas import tpu_sc as plsc

N = 10000
E = 320000
F = 128
NC, NS, L = 2, 16, 16
NW = NC * NS
E_PER_W = E // NW
CHUNK = 80
N_CHUNKS = E_PER_W // CHUNK
NPAD = 10240
RPT = NPAD // NS

_MESH = plsc.VectorSubcoreMesh(
    core_axis_name="c", subcore_axis_name="s", num_cores=NC, num_subcores=NS)
_SC_PARAMS = pltpu.CompilerParams(needs_layout_passes=False)


def _dense1_body(x_ref, w_ref, a_ref, z_ref, st_ref):
    z = jnp.dot(x_ref[...], w_ref[...], preferred_element_type=jnp.float32)
    z_ref[...] = z
    aa = jnp.concatenate([a_ref[:F, :], a_ref[F:, :]], axis=1)
    st_ref[...] = jnp.dot(z, aa, preferred_element_type=jnp.float32)


def _dense1(x, W1, a1):
    return pl.pallas_call(
        _dense1_body,
        out_shape=[jax.ShapeDtypeStruct((N, F), jnp.float32),
                   jax.ShapeDtypeStruct((N, 2), jnp.float32)],
    )(x, W1, a1)


def _sc1_body(st_hbm, src_hbm, dst_hbm, z_hbm, zrow_hbm, zvec_hbm,
              acc_out, den_out,
              acc_sh, den_sh, st_t, srcb, dstb, exb, rows, gsem):
    c = lax.axis_index("c")
    s = lax.axis_index("s")
    wid = c * NS + s
    row0 = s * RPT
    pltpu.sync_copy(zrow_hbm, acc_sh.at[pl.ds(row0, RPT)])
    pltpu.sync_copy(zvec_hbm, den_sh.at[pl.ds(row0, RPT)])
    pltpu.sync_copy(st_hbm, st_t)
    plsc.subcore_barrier()

    base = wid * E_PER_W

    def chunk_body(i, carry):
        off = base + i * CHUNK
        pltpu.sync_copy(src_hbm.at[pl.ds(off, CHUNK)], srcb)
        pltpu.sync_copy(dst_hbm.at[pl.ds(off, CHUNK)], dstb)
        gather = pltpu.async_copy(z_hbm.at[srcb], rows, gsem)
        for g in range(CHUNK // L):
            sl = pl.ds(g * L, L)
            si = srcb[sl]
            di = dstb[sl]
            sv = plsc.load_gather(st_t, [si * 2])
            tv = plsc.load_gather(st_t, [di * 2 + 1])
            e = sv + tv
            e = jnp.where(e >= 0.0, e, 0.2 * e)
            exb[sl] = jnp.exp(e)
        pltpu.sync_copy(exb, den_sh.at[dstb], add=True)
        gather.wait()

        def scale_body(j, carry2):
            m = plsc.load_gather(exb, [jnp.full((L,), j, jnp.int32)])
            for v in range(F // L):
                fs = pl.ds(v * L, L)
                rows[j, fs] = rows[j, fs] * m
            return carry2

        lax.fori_loop(0, CHUNK, scale_body, 0)
        pltpu.sync_copy(rows, acc_sh.at[dstb], add=True)
        return carry

    lax.fori_loop(0, N_CHUNKS, chunk_body, 0)
    plsc.subcore_barrier()
    pltpu.sync_copy(acc_sh.at[pl.ds(row0, RPT)],
                    acc_out.at[c, pl.ds(row0, RPT)])
    pltpu.sync_copy(den_sh.at[pl.ds(row0, RPT)],
                    den_out.at[c, pl.ds(row0, RPT)])


def _sc1(st_flat, src, dst, z1, zrow, zvec):
    return pl.kernel(
        _sc1_body,
        out_type=[jax.ShapeDtypeStruct((NC, NPAD, F), jnp.float32),
                  jax.ShapeDtypeStruct((NC, NPAD), jnp.float32)],
        mesh=_MESH,
        scratch_types=[
            pltpu.VMEM_SHARED((NPAD, F), jnp.float32),
            pltpu.VMEM_SHARED((NPAD,), jnp.float32),
            pltpu.VMEM((2 * N,), jnp.float32),
            pltpu.VMEM((CHUNK,), jnp.int32),
            pltpu.VMEM((CHUNK,), jnp.int32),
            pltpu.VMEM((CHUNK,), jnp.float32),
            pltpu.VMEM((CHUNK, F), jnp.float32),
            pltpu.SemaphoreType.DMA,
        ],
        compiler_params=_SC_PARAMS,
    )(st_flat, src, dst, z1, zrow, zvec)


def _dense2_body(accp_ref, denp_ref, w2_ref, a2_ref, t3_ref):
    acc = accp_ref[0, :N, :] + accp_ref[1, :N, :]
    den = denp_ref[0, :N] + denp_ref[1, :N]
    rden = 1.0 / (den + 1e-9)
    h = jnp.maximum(acc * rden[:, None], 0.0)
    z2 = jnp.dot(h, w2_ref[...], preferred_element_type=jnp.float32)
    coef = jnp.concatenate(
        [jnp.ones((1, 1), jnp.float32), a2_ref[0:1, :], a2_ref[1:2, :]],
        axis=1)
    t3_ref[...] = z2 * coef


def _dense2(accp, denp, W2, a2):
    return pl.pallas_call(
        _dense2_body,
        out_shape=jax.ShapeDtypeStruct((N, 3), jnp.float32),
    )(accp, denp, W2, a2)


def _sc2_body(t3_hbm, src_hbm, dst_hbm, zvec_hbm,
              num_out, den_out,
              num_sh, den_sh, t3_t, srcb, dstb, exb, numb):
    c = lax.axis_index("c")
    s = lax.axis_index("s")
    wid = c * NS + s
    row0 = s * RPT
    pltpu.sync_copy(zvec_hbm, num_sh.at[pl.ds(row0, RPT)])
    pltpu.sync_copy(zvec_hbm, den_sh.at[pl.ds(row0, RPT)])
    pltpu.sync_copy(t3_hbm, t3_t)
    plsc.subcore_barrier()

    base = wid * E_PER_W

    def chunk_body(i, carry):
        off = base + i * CHUNK
        pltpu.sync_copy(src_hbm.at[pl.ds(off, CHUNK)], srcb)
        pltpu.sync_copy(dst_hbm.at[pl.ds(off, CHUNK)], dstb)
        for g in range(CHUNK // L):
            sl = pl.ds(g * L, L)
            si = srcb[sl]
            di = dstb[sl]
            zs = plsc.load_gather(t3_t, [si * 3])
            ss = plsc.load_gather(t3_t, [si * 3 + 1])
            td = plsc.load_gather(t3_t, [di * 3 + 2])
            e = ss + td
            e = jnp.where(e >= 0.0, e, 0.2 * e)
            ex = jnp.exp(e)
            exb[sl] = ex
            numb[sl] = ex * zs
        pltpu.sync_copy(exb, den_sh.at[dstb], add=True)
        pltpu.sync_copy(numb, num_sh.at[dstb], add=True)
        return carry

    lax.fori_loop(0, N_CHUNKS, chunk_body, 0)
    plsc.subcore_barrier()
    pltpu.sync_copy(num_sh.at[pl.ds(row0, RPT)],
                    num_out.at[c, pl.ds(row0, RPT)])
    pltpu.sync_copy(den_sh.at[pl.ds(row0, RPT)],
                    den_out.at[c, pl.ds(row0, RPT)])


def _sc2(t3_flat, src, dst, zvec):
    return pl.kernel(
        _sc2_body,
        out_type=[jax.ShapeDtypeStruct((NC, NPAD), jnp.float32),
                  jax.ShapeDtypeStruct((NC, NPAD), jnp.float32)],
        mesh=_MESH,
        scratch_types=[
            pltpu.VMEM_SHARED((NPAD,), jnp.float32),
            pltpu.VMEM_SHARED((NPAD,), jnp.float32),
            pltpu.VMEM((3 * N,), jnp.float32),
            pltpu.VMEM((CHUNK,), jnp.int32),
            pltpu.VMEM((CHUNK,), jnp.int32),
            pltpu.VMEM((CHUNK,), jnp.float32),
            pltpu.VMEM((CHUNK,), jnp.float32),
        ],
        compiler_params=_SC_PARAMS,
    )(t3_flat, src, dst, zvec)


def _final_body(nump_ref, denp_ref, out_ref):
    num = nump_ref[0, :N] + nump_ref[1, :N]
    den = denp_ref[0, :N] + denp_ref[1, :N]
    r = num / (den + 1e-9)
    out_ref[...] = jax.nn.sigmoid(r)[:, None]


def _final(nump, denp):
    return pl.pallas_call(
        _final_body,
        out_shape=jax.ShapeDtypeStruct((N, 1), jnp.float32),
    )(nump, denp)


def kernel(edge_index, x, W1, a1, W2, a2):
    ei = edge_index.astype(jnp.int32)
    src = ei[0]
    dst = ei[1]
    zrow = jnp.zeros((RPT, F), jnp.float32)
    zvec = jnp.zeros((RPT,), jnp.float32)

    z1, st1 = _dense1(x, W1, a1)
    accp, denp = _sc1(st1.reshape(2 * N), src, dst, z1, zrow, zvec)
    t3 = _dense2(accp, denp, W2, a2)
    nump, denp2 = _sc2(t3.reshape(3 * N), src, dst, zvec)
    return _final(nump, denp2)


# speedup vs baseline: 34.6774x; 1.2340x over previous
import jax
import jax.numpy as jnp
from jax import lax
from jax.experimental import pallas as pl
from jax.experimental.pallas import tpu as pltpu
from jax.experimental.pallas import tpu_sc as plsc

N = 10000
E = 320000
F = 128
NC, NS, L = 2, 16, 16
NW = NC * NS
E_PER_W = E // NW
CHUNK = 80
N_CHUNKS = E_PER_W // CHUNK
NPAD = 10240
RPT = NPAD // NS

_MESH = plsc.VectorSubcoreMesh(
    core_axis_name="c", subcore_axis_name="s", num_cores=NC, num_subcores=NS)
_SC_PARAMS = pltpu.CompilerParams(needs_layout_passes=False)


def _dense1_body(x_ref, w_ref, a_ref, z_ref, st_ref):
    z = jnp.dot(x_ref[...], w_ref[...], preferred_element_type=jnp.float32)
    z_ref[...] = z
    aa = jnp.concatenate([a_ref[:F, :], a_ref[F:, :]], axis=1)
    st_ref[...] = jnp.dot(z, aa, preferred_element_type=jnp.float32)


def _dense1(x, W1, a1):
    return pl.pallas_call(
        _dense1_body,
        out_shape=[jax.ShapeDtypeStruct((N, F), jnp.float32),
                   jax.ShapeDtypeStruct((N, 2), jnp.float32)],
    )(x, W1, a1)


def _sc1_body(st_hbm, src_hbm, dst_hbm, z_hbm, zrow_hbm, zvec_hbm,
              acc_out, den_out, *sc):
    (acc_sh, den_sh, st_sh) = sc[0:3]
    SRC = sc[3:7]
    DST = sc[7:11]
    SIDX = sc[11:13]
    DIDX = sc[13:15]
    EX = sc[15:17]
    SV = sc[17:19]
    TV = sc[19:21]
    G = sc[21:23]
    S = sc[23:25]
    ISEM = sc[25:29]
    GSEM = sc[29:31]
    SSEM = sc[31:33]
    XSEM = sc[33:35]
    TSEM = sc[35:37]

    c = lax.axis_index("c")
    s = lax.axis_index("s")
    wid = c * NS + s
    row0 = s * RPT
    base = wid * E_PER_W
    pltpu.sync_copy(zrow_hbm, acc_sh.at[pl.ds(row0, RPT)])
    pltpu.sync_copy(zvec_hbm, den_sh.at[pl.ds(row0, RPT)])

    @pl.when(s == 0)
    def _():
        pltpu.sync_copy(st_hbm, st_sh)

    plsc.subcore_barrier()

    def start_idx(j, q):
        off = base + j * CHUNK
        pltpu.async_copy(src_hbm.at[pl.ds(off, CHUNK)], SRC[q], ISEM[q])
        pltpu.async_copy(dst_hbm.at[pl.ds(off, CHUNK)], DST[q], ISEM[q])

    def wait_idx(j, q):
        off = base + j * CHUNK
        pltpu.make_async_copy(src_hbm.at[pl.ds(off, CHUNK)], SRC[q],
                              ISEM[q]).wait()
        pltpu.make_async_copy(dst_hbm.at[pl.ds(off, CHUNK)], DST[q],
                              ISEM[q]).wait()

    def prep_st(b, q):
        for g in range(CHUNK // L):
            sl = pl.ds(g * L, L)
            SIDX[b][sl] = SRC[q][sl] * 2
            DIDX[b][sl] = DST[q][sl] * 2 + 1
        pltpu.async_copy(st_sh.at[SIDX[b]], SV[b], TSEM[b])
        pltpu.async_copy(st_sh.at[DIDX[b]], TV[b], TSEM[b])

    def wait_st(b):
        pltpu.make_async_copy(st_sh.at[SIDX[b]], SV[b], TSEM[b]).wait()
        pltpu.make_async_copy(st_sh.at[DIDX[b]], TV[b], TSEM[b]).wait()

    def start_gather(b, q):
        pltpu.async_copy(z_hbm.at[SRC[q]], G[b], GSEM[b])

    def wait_gather(b, q):
        pltpu.make_async_copy(z_hbm.at[SRC[q]], G[b], GSEM[b]).wait()

    def start_scatter(b, q):
        pltpu.async_copy(S[b], acc_sh.at[DST[q]], SSEM[b], add=True)
        pltpu.async_copy(EX[b], den_sh.at[DST[q]], XSEM[b], add=True)

    def wait_scatter(b, q):
        pltpu.make_async_copy(S[b], acc_sh.at[DST[q]], SSEM[b]).wait()
        pltpu.make_async_copy(EX[b], den_sh.at[DST[q]], XSEM[b]).wait()

    def compute_ex(b):
        for g in range(CHUNK // L):
            sl = pl.ds(g * L, L)
            e = SV[b][sl] + TV[b][sl]
            e = jnp.where(e >= 0.0, e, 0.2 * e)
            EX[b][sl] = jnp.exp(e)

    def scale(b):
        def scale_body(i, carry2):
            m = plsc.load_gather(EX[b], [jnp.full((L,), i, jnp.int32)])
            for v in range(F // L):
                fs = pl.ds(v * L, L)
                S[b][i, fs] = G[b][i, fs] * m
            return carry2

        lax.fori_loop(0, CHUNK, scale_body, 0)

    start_idx(0, 0)
    start_idx(1, 1)
    wait_idx(0, 0)
    prep_st(0, 0)
    start_gather(0, 0)
    wait_idx(1, 1)
    prep_st(1, 1)
    start_gather(1, 1)
    start_idx(2, 2)
    start_idx(3, 3)
    for j0 in (0, 1):
        b = q = j0
        wait_st(b)
        compute_ex(b)
        wait_gather(b, q)
        scale(b)
        start_scatter(b, q)
        wait_idx(j0 + 2, j0 + 2)
        prep_st(b, j0 + 2)
        start_gather(b, j0 + 2)

    def ring_body(i, carry):
        for b4 in range(4):
            j = 4 * i + 2 + b4
            b = b4 % 2
            q = (2 + b4) % 4
            qn = (q + 2) % 4
            wait_scatter(b, q)
            start_idx(j + 2, qn)
            wait_st(b)
            compute_ex(b)
            wait_gather(b, q)
            scale(b)
            start_scatter(b, q)
            wait_idx(j + 2, qn)
            prep_st(b, qn)
            start_gather(b, qn)
        return carry

    lax.fori_loop(0, 30, ring_body, 0)

    jt = jnp.int32(122)
    wait_scatter(0, 2)
    start_idx(jt + 2, 0)
    wait_st(0)
    compute_ex(0)
    wait_gather(0, 2)
    scale(0)
    start_scatter(0, 2)
    wait_idx(jt + 2, 0)
    prep_st(0, 0)
    start_gather(0, 0)
    wait_scatter(1, 3)
    wait_st(1)
    compute_ex(1)
    wait_gather(1, 3)
    scale(1)
    start_scatter(1, 3)
    wait_scatter(0, 0)
    wait_st(0)
    compute_ex(0)
    wait_gather(0, 0)
    scale(0)
    start_scatter(0, 0)
    wait_scatter(1, 3)
    wait_scatter(0, 0)

    plsc.subcore_barrier()
    pltpu.sync_copy(acc_sh.at[pl.ds(row0, RPT)],
                    acc_out.at[c, pl.ds(row0, RPT)])
    pltpu.sync_copy(den_sh.at[pl.ds(row0, RPT)],
                    den_out.at[c, pl.ds(row0, RPT)])


def _sc1(st_flat, src, dst, z1, zrow, zvec):
    return pl.kernel(
        _sc1_body,
        out_type=[jax.ShapeDtypeStruct((NC, NPAD, F), jnp.float32),
                  jax.ShapeDtypeStruct((NC, NPAD), jnp.float32)],
        mesh=_MESH,
        scratch_types=[
            pltpu.VMEM_SHARED((NPAD, F), jnp.float32),
            pltpu.VMEM_SHARED((NPAD,), jnp.float32),
            pltpu.VMEM_SHARED((2 * N,), jnp.float32),
        ] + [pltpu.VMEM((CHUNK,), jnp.int32)] * 12
          + [pltpu.VMEM((CHUNK,), jnp.float32)] * 6
          + [pltpu.VMEM((CHUNK, F), jnp.float32)] * 4
          + [pltpu.SemaphoreType.DMA] * 12,
        compiler_params=_SC_PARAMS,
    )(st_flat, src, dst, z1, zrow, zvec)


def _dense2_body(accp_ref, denp_ref, w2_ref, a2_ref, t3_ref):
    acc = accp_ref[0, :N, :] + accp_ref[1, :N, :]
    den = denp_ref[0, :N] + denp_ref[1, :N]
    rden = 1.0 / (den + 1e-9)
    h = jnp.maximum(acc * rden[:, None], 0.0)
    z2 = jnp.dot(h, w2_ref[...], preferred_element_type=jnp.float32)
    coef = jnp.concatenate(
        [jnp.ones((1, 1), jnp.float32), a2_ref[0:1, :], a2_ref[1:2, :]],
        axis=1)
    t3_ref[...] = z2 * coef


def _dense2(accp, denp, W2, a2):
    return pl.pallas_call(
        _dense2_body,
        out_shape=jax.ShapeDtypeStruct((N, 3), jnp.float32),
    )(accp, denp, W2, a2)


def _sc2_body(t3_hbm, src_hbm, dst_hbm, zvec_hbm,
              num_out, den_out, *sc):
    (num_sh, den_sh, t3_t) = sc[0:3]
    SRC = sc[3:7]
    DST = sc[7:11]
    EX = sc[11:13]
    NUM = sc[13:15]
    ISEM = sc[15:19]
    XSEM = sc[19:21]
    YSEM = sc[21:23]

    c = lax.axis_index("c")
    s = lax.axis_index("s")
    wid = c * NS + s
    row0 = s * RPT
    base = wid * E_PER_W
    pltpu.sync_copy(zvec_hbm, num_sh.at[pl.ds(row0, RPT)])
    pltpu.sync_copy(zvec_hbm, den_sh.at[pl.ds(row0, RPT)])
    pltpu.sync_copy(t3_hbm, t3_t)
    plsc.subcore_barrier()

    def start_idx(j, q):
        off = base + j * CHUNK
        pltpu.async_copy(src_hbm.at[pl.ds(off, CHUNK)], SRC[q], ISEM[q])
        pltpu.async_copy(dst_hbm.at[pl.ds(off, CHUNK)], DST[q], ISEM[q])

    def wait_idx(j, q):
        off = base + j * CHUNK
        pltpu.make_async_copy(src_hbm.at[pl.ds(off, CHUNK)], SRC[q],
                              ISEM[q]).wait()
        pltpu.make_async_copy(dst_hbm.at[pl.ds(off, CHUNK)], DST[q],
                              ISEM[q]).wait()

    def compute(b, q):
        for g in range(CHUNK // L):
            sl = pl.ds(g * L, L)
            si = SRC[q][sl]
            di = DST[q][sl]
            zs = plsc.load_gather(t3_t, [si * 3])
            ss = plsc.load_gather(t3_t, [si * 3 + 1])
            td = plsc.load_gather(t3_t, [di * 3 + 2])
            e = ss + td
            e = jnp.where(e >= 0.0, e, 0.2 * e)
            ex = jnp.exp(e)
            EX[b][sl] = ex
            NUM[b][sl] = ex * zs

    def start_scatter(b, q):
        pltpu.async_copy(EX[b], den_sh.at[DST[q]], XSEM[b], add=True)
        pltpu.async_copy(NUM[b], num_sh.at[DST[q]], YSEM[b], add=True)

    def wait_scatter(b, q):
        pltpu.make_async_copy(EX[b], den_sh.at[DST[q]], XSEM[b]).wait()
        pltpu.make_async_copy(NUM[b], num_sh.at[DST[q]], YSEM[b]).wait()

    start_idx(0, 0)
    start_idx(1, 1)
    start_idx(2, 2)
    start_idx(3, 3)
    for j0 in (0, 1):
        wait_idx(j0, j0)
        compute(j0, j0)
        start_scatter(j0, j0)

    def ring_body(i, carry):
        for b4 in range(4):
            j = 4 * i + 2 + b4
            b = b4 % 2
            q = (2 + b4) % 4
            qn = (q + 2) % 4
            wait_scatter(b, q)
            start_idx(j + 2, qn)
            wait_idx(j, q)
            compute(b, q)
            start_scatter(b, q)
        return carry

    lax.fori_loop(0, 30, ring_body, 0)

    jt = jnp.int32(122)
    wait_scatter(0, 2)
    start_idx(jt + 2, 0)
    wait_idx(jt, 2)
    compute(0, 2)
    start_scatter(0, 2)
    wait_scatter(1, 3)
    wait_idx(jt + 1, 3)
    compute(1, 3)
    start_scatter(1, 3)
    wait_scatter(0, 0)
    wait_idx(jt + 2, 0)
    compute(0, 0)
    start_scatter(0, 0)
    wait_scatter(1, 3)
    wait_scatter(0, 0)
    plsc.subcore_barrier()
    pltpu.sync_copy(num_sh.at[pl.ds(row0, RPT)],
                    num_out.at[c, pl.ds(row0, RPT)])
    pltpu.sync_copy(den_sh.at[pl.ds(row0, RPT)],
                    den_out.at[c, pl.ds(row0, RPT)])


def _sc2(t3_flat, src, dst, zvec):
    return pl.kernel(
        _sc2_body,
        out_type=[jax.ShapeDtypeStruct((NC, NPAD), jnp.float32),
                  jax.ShapeDtypeStruct((NC, NPAD), jnp.float32)],
        mesh=_MESH,
        scratch_types=[
            pltpu.VMEM_SHARED((NPAD,), jnp.float32),
            pltpu.VMEM_SHARED((NPAD,), jnp.float32),
            pltpu.VMEM((3 * N,), jnp.float32),
        ] + [pltpu.VMEM((CHUNK,), jnp.int32)] * 8
          + [pltpu.VMEM((CHUNK,), jnp.float32)] * 4
          + [pltpu.SemaphoreType.DMA] * 8,
        compiler_params=_SC_PARAMS,
    )(t3_flat, src, dst, zvec)


def _final_body(nump_ref, denp_ref, out_ref):
    num = nump_ref[0, :N] + nump_ref[1, :N]
    den = denp_ref[0, :N] + denp_ref[1, :N]
    r = num / (den + 1e-9)
    out_ref[...] = jax.nn.sigmoid(r)[:, None]


def _final(nump, denp):
    return pl.pallas_call(
        _final_body,
        out_shape=jax.ShapeDtypeStruct((N, 1), jnp.float32),
    )(nump, denp)


def kernel(edge_index, x, W1, a1, W2, a2):
    ei = edge_index.astype(jnp.int32)
    src = ei[0]
    dst = ei[1]
    zrow = jnp.zeros((RPT, F), jnp.float32)
    zvec = jnp.zeros((RPT,), jnp.float32)

    z1, st1 = _dense1(x, W1, a1)
    accp, denp = _sc1(st1.reshape(2 * N), src, dst, z1, zrow, zvec)
    t3 = _dense2(accp, denp, W2, a2)
    nump, denp2 = _sc2(t3.reshape(3 * N), src, dst, zvec)
    return _final(nump, denp2)


# speedup vs baseline: 34.8587x; 1.0052x over previous
import jax
import jax.numpy as jnp
from jax import lax
from jax.experimental import pallas as pl
from jax.experimental.pallas import tpu as pltpu
from jax.experimental.pallas import tpu_sc as plsc

N = 10000
E = 320000
F = 128
NC, NS, L = 2, 16, 16
NW = NC * NS
E_PER_W = E // NW
CHUNK = 80
N_CHUNKS = E_PER_W // CHUNK
NPAD = 10240
RPT = NPAD // NS

_MESH = plsc.VectorSubcoreMesh(
    core_axis_name="c", subcore_axis_name="s", num_cores=NC, num_subcores=NS)
_SC_PARAMS = pltpu.CompilerParams(needs_layout_passes=False)


def _dense1_body(x_ref, w_ref, a_ref, z_ref, st_ref):
    z = jnp.dot(x_ref[...], w_ref[...], preferred_element_type=jnp.float32)
    z_ref[...] = z
    aa = jnp.concatenate([a_ref[:F, :], a_ref[F:, :]], axis=1)
    st_ref[...] = jnp.dot(z, aa, preferred_element_type=jnp.float32)


def _dense1(x, W1, a1):
    return pl.pallas_call(
        _dense1_body,
        out_shape=[jax.ShapeDtypeStruct((N, F), jnp.float32),
                   jax.ShapeDtypeStruct((N, 2), jnp.float32)],
    )(x, W1, a1)


def _sc1_body(st_hbm, src_hbm, dst_hbm, z_hbm, zrow_hbm, zvec_hbm,
              acc_out, den_out, *sc):
    (acc_sh, den_sh, st_sh) = sc[0:3]
    SRC = sc[3:7]
    DST = sc[7:11]
    SIDX = sc[11:13]
    DIDX = sc[13:15]
    EX = sc[15:17]
    SV = sc[17:19]
    TV = sc[19:21]
    G = sc[21:23]
    S = sc[23:25]
    ISEM = sc[25:29]
    GSEM = sc[29:31]
    SSEM = sc[31:33]
    XSEM = sc[33:35]
    TSEM = sc[35:37]

    c = lax.axis_index("c")
    s = lax.axis_index("s")
    wid = c * NS + s
    row0 = s * RPT
    base = wid * E_PER_W
    pltpu.sync_copy(zrow_hbm, acc_sh.at[pl.ds(row0, RPT)])
    pltpu.sync_copy(zvec_hbm, den_sh.at[pl.ds(row0, RPT)])

    @pl.when(s == 0)
    def _():
        pltpu.sync_copy(st_hbm, st_sh)

    plsc.subcore_barrier()

    def start_idx(j, q):
        off = base + j * CHUNK
        pltpu.async_copy(src_hbm.at[pl.ds(off, CHUNK)], SRC[q], ISEM[q])
        pltpu.async_copy(dst_hbm.at[pl.ds(off, CHUNK)], DST[q], ISEM[q])

    def wait_idx(j, q):
        off = base + j * CHUNK
        pltpu.make_async_copy(src_hbm.at[pl.ds(off, CHUNK)], SRC[q],
                              ISEM[q]).wait()
        pltpu.make_async_copy(dst_hbm.at[pl.ds(off, CHUNK)], DST[q],
                              ISEM[q]).wait()

    def prep_st(b, q):
        for g in range(CHUNK // L):
            sl = pl.ds(g * L, L)
            SIDX[b][sl] = SRC[q][sl] * 2
            DIDX[b][sl] = DST[q][sl] * 2 + 1
        pltpu.async_copy(st_sh.at[SIDX[b]], SV[b], TSEM[b])
        pltpu.async_copy(st_sh.at[DIDX[b]], TV[b], TSEM[b])

    def wait_st(b):
        pltpu.make_async_copy(st_sh.at[SIDX[b]], SV[b], TSEM[b]).wait()
        pltpu.make_async_copy(st_sh.at[DIDX[b]], TV[b], TSEM[b]).wait()

    def start_gather(b, q):
        pltpu.async_copy(z_hbm.at[SRC[q]], G[b], GSEM[b])

    def wait_gather(b, q):
        pltpu.make_async_copy(z_hbm.at[SRC[q]], G[b], GSEM[b]).wait()

    def start_scatter(b, q):
        pltpu.async_copy(S[b], acc_sh.at[DST[q]], SSEM[b], add=True)
        pltpu.async_copy(EX[b], den_sh.at[DST[q]], XSEM[b], add=True)

    def wait_scatter(b, q):
        pltpu.make_async_copy(S[b], acc_sh.at[DST[q]], SSEM[b]).wait()
        pltpu.make_async_copy(EX[b], den_sh.at[DST[q]], XSEM[b]).wait()

    def compute_ex(b):
        for g in range(CHUNK // L):
            sl = pl.ds(g * L, L)
            e = SV[b][sl] + TV[b][sl]
            e = jnp.where(e >= 0.0, e, 0.2 * e)
            EX[b][sl] = jnp.exp(e)

    def scale(b):
        def scale_body(i8, carry2):
            i0 = i8 * 8
            for u in range(8):
                i = i0 + u
                m = plsc.load_gather(EX[b], [jnp.full((L,), i, jnp.int32)])
                for v in range(F // L):
                    fs = pl.ds(v * L, L)
                    S[b][i, fs] = G[b][i, fs] * m
            return carry2

        lax.fori_loop(0, CHUNK // 8, scale_body, 0)

    start_idx(0, 0)
    start_idx(1, 1)
    wait_idx(0, 0)
    prep_st(0, 0)
    start_gather(0, 0)
    wait_idx(1, 1)
    prep_st(1, 1)
    start_gather(1, 1)
    start_idx(2, 2)
    start_idx(3, 3)
    for j0 in (0, 1):
        b = q = j0
        wait_st(b)
        compute_ex(b)
        wait_gather(b, q)
        scale(b)
        start_scatter(b, q)
        wait_idx(j0 + 2, j0 + 2)
        prep_st(b, j0 + 2)
        start_gather(b, j0 + 2)

    def ring_body(i, carry):
        for b4 in range(4):
            j = 4 * i + 2 + b4
            b = b4 % 2
            q = (2 + b4) % 4
            qn = (q + 2) % 4
            wait_scatter(b, q)
            start_idx(j + 2, qn)
            wait_st(b)
            compute_ex(b)
            wait_gather(b, q)
            scale(b)
            start_scatter(b, q)
            wait_idx(j + 2, qn)
            prep_st(b, qn)
            start_gather(b, qn)
        return carry

    lax.fori_loop(0, 30, ring_body, 0)

    jt = jnp.int32(122)
    wait_scatter(0, 2)
    start_idx(jt + 2, 0)
    wait_st(0)
    compute_ex(0)
    wait_gather(0, 2)
    scale(0)
    start_scatter(0, 2)
    wait_idx(jt + 2, 0)
    prep_st(0, 0)
    start_gather(0, 0)
    wait_scatter(1, 3)
    wait_st(1)
    compute_ex(1)
    wait_gather(1, 3)
    scale(1)
    start_scatter(1, 3)
    wait_scatter(0, 0)
    wait_st(0)
    compute_ex(0)
    wait_gather(0, 0)
    scale(0)
    start_scatter(0, 0)
    wait_scatter(1, 3)
    wait_scatter(0, 0)

    plsc.subcore_barrier()
    pltpu.sync_copy(acc_sh.at[pl.ds(row0, RPT)],
                    acc_out.at[c, pl.ds(row0, RPT)])
    pltpu.sync_copy(den_sh.at[pl.ds(row0, RPT)],
                    den_out.at[c, pl.ds(row0, RPT)])


def _sc1(st_flat, src, dst, z1, zrow, zvec):
    return pl.kernel(
        _sc1_body,
        out_type=[jax.ShapeDtypeStruct((NC, NPAD, F), jnp.float32),
                  jax.ShapeDtypeStruct((NC, NPAD), jnp.float32)],
        mesh=_MESH,
        scratch_types=[
            pltpu.VMEM_SHARED((NPAD, F), jnp.float32),
            pltpu.VMEM_SHARED((NPAD,), jnp.float32),
            pltpu.VMEM_SHARED((2 * N,), jnp.float32),
        ] + [pltpu.VMEM((CHUNK,), jnp.int32)] * 12
          + [pltpu.VMEM((CHUNK,), jnp.float32)] * 6
          + [pltpu.VMEM((CHUNK, F), jnp.float32)] * 4
          + [pltpu.SemaphoreType.DMA] * 12,
        compiler_params=_SC_PARAMS,
    )(st_flat, src, dst, z1, zrow, zvec)


def _dense2_body(accp_ref, denp_ref, w2_ref, a2_ref, t3_ref):
    acc = accp_ref[0, :N, :] + accp_ref[1, :N, :]
    den = denp_ref[0, :N] + denp_ref[1, :N]
    rden = 1.0 / (den + 1e-9)
    h = jnp.maximum(acc * rden[:, None], 0.0)
    z2 = jnp.dot(h, w2_ref[...], preferred_element_type=jnp.float32)
    coef = jnp.concatenate(
        [jnp.ones((1, 1), jnp.float32), a2_ref[0:1, :], a2_ref[1:2, :]],
        axis=1)
    t3_ref[...] = z2 * coef


def _dense2(accp, denp, W2, a2):
    return pl.pallas_call(
        _dense2_body,
        out_shape=jax.ShapeDtypeStruct((N, 3), jnp.float32),
    )(accp, denp, W2, a2)


def _sc2_body(t3_hbm, src_hbm, dst_hbm, zvec_hbm,
              num_out, den_out, *sc):
    (num_sh, den_sh, t3_t) = sc[0:3]
    SRC = sc[3:7]
    DST = sc[7:11]
    EX = sc[11:13]
    NUM = sc[13:15]
    ISEM = sc[15:19]
    XSEM = sc[19:21]
    YSEM = sc[21:23]

    c = lax.axis_index("c")
    s = lax.axis_index("s")
    wid = c * NS + s
    row0 = s * RPT
    base = wid * E_PER_W
    pltpu.sync_copy(zvec_hbm, num_sh.at[pl.ds(row0, RPT)])
    pltpu.sync_copy(zvec_hbm, den_sh.at[pl.ds(row0, RPT)])
    pltpu.sync_copy(t3_hbm, t3_t)
    plsc.subcore_barrier()

    def start_idx(j, q):
        off = base + j * CHUNK
        pltpu.async_copy(src_hbm.at[pl.ds(off, CHUNK)], SRC[q], ISEM[q])
        pltpu.async_copy(dst_hbm.at[pl.ds(off, CHUNK)], DST[q], ISEM[q])

    def wait_idx(j, q):
        off = base + j * CHUNK
        pltpu.make_async_copy(src_hbm.at[pl.ds(off, CHUNK)], SRC[q],
                              ISEM[q]).wait()
        pltpu.make_async_copy(dst_hbm.at[pl.ds(off, CHUNK)], DST[q],
                              ISEM[q]).wait()

    def compute(b, q):
        for g in range(CHUNK // L):
            sl = pl.ds(g * L, L)
            si = SRC[q][sl]
            di = DST[q][sl]
            zs = plsc.load_gather(t3_t, [si * 3])
            ss = plsc.load_gather(t3_t, [si * 3 + 1])
            td = plsc.load_gather(t3_t, [di * 3 + 2])
            e = ss + td
            e = jnp.where(e >= 0.0, e, 0.2 * e)
            ex = jnp.exp(e)
            EX[b][sl] = ex
            NUM[b][sl] = ex * zs

    def start_scatter(b, q):
        pltpu.async_copy(EX[b], den_sh.at[DST[q]], XSEM[b], add=True)
        pltpu.async_copy(NUM[b], num_sh.at[DST[q]], YSEM[b], add=True)

    def wait_scatter(b, q):
        pltpu.make_async_copy(EX[b], den_sh.at[DST[q]], XSEM[b]).wait()
        pltpu.make_async_copy(NUM[b], num_sh.at[DST[q]], YSEM[b]).wait()

    start_idx(0, 0)
    start_idx(1, 1)
    start_idx(2, 2)
    start_idx(3, 3)
    for j0 in (0, 1):
        wait_idx(j0, j0)
        compute(j0, j0)
        start_scatter(j0, j0)

    def ring_body(i, carry):
        for b4 in range(4):
            j = 4 * i + 2 + b4
            b = b4 % 2
            q = (2 + b4) % 4
            qn = (q + 2) % 4
            wait_scatter(b, q)
            start_idx(j + 2, qn)
            wait_idx(j, q)
            compute(b, q)
            start_scatter(b, q)
        return carry

    lax.fori_loop(0, 30, ring_body, 0)

    jt = jnp.int32(122)
    wait_scatter(0, 2)
    start_idx(jt + 2, 0)
    wait_idx(jt, 2)
    compute(0, 2)
    start_scatter(0, 2)
    wait_scatter(1, 3)
    wait_idx(jt + 1, 3)
    compute(1, 3)
    start_scatter(1, 3)
    wait_scatter(0, 0)
    wait_idx(jt + 2, 0)
    compute(0, 0)
    start_scatter(0, 0)
    wait_scatter(1, 3)
    wait_scatter(0, 0)
    plsc.subcore_barrier()
    pltpu.sync_copy(num_sh.at[pl.ds(row0, RPT)],
                    num_out.at[c, pl.ds(row0, RPT)])
    pltpu.sync_copy(den_sh.at[pl.ds(row0, RPT)],
                    den_out.at[c, pl.ds(row0, RPT)])


def _sc2(t3_flat, src, dst, zvec):
    return pl.kernel(
        _sc2_body,
        out_type=[jax.ShapeDtypeStruct((NC, NPAD), jnp.float32),
                  jax.ShapeDtypeStruct((NC, NPAD), jnp.float32)],
        mesh=_MESH,
        scratch_types=[
            pltpu.VMEM_SHARED((NPAD,), jnp.float32),
            pltpu.VMEM_SHARED((NPAD,), jnp.float32),
            pltpu.VMEM((3 * N,), jnp.float32),
        ] + [pltpu.VMEM((CHUNK,), jnp.int32)] * 8
          + [pltpu.VMEM((CHUNK,), jnp.float32)] * 4
          + [pltpu.SemaphoreType.DMA] * 8,
        compiler_params=_SC_PARAMS,
    )(t3_flat, src, dst, zvec)


def _final_body(nump_ref, denp_ref, out_ref):
    num = nump_ref[0, :N] + nump_ref[1, :N]
    den = denp_ref[0, :N] + denp_ref[1, :N]
    r = num / (den + 1e-9)
    out_ref[...] = jax.nn.sigmoid(r)[:, None]


def _final(nump, denp):
    return pl.pallas_call(
        _final_body,
        out_shape=jax.ShapeDtypeStruct((N, 1), jnp.float32),
    )(nump, denp)


def kernel(edge_index, x, W1, a1, W2, a2):
    ei = edge_index.astype(jnp.int32)
    src = ei[0]
    dst = ei[1]
    zrow = jnp.zeros((RPT, F), jnp.float32)
    zvec = jnp.zeros((RPT,), jnp.float32)

    z1, st1 = _dense1(x, W1, a1)
    accp, denp = _sc1(st1.reshape(2 * N), src, dst, z1, zrow, zvec)
    t3 = _dense2(accp, denp, W2, a2)
    nump, denp2 = _sc2(t3.reshape(3 * N), src, dst, zvec)
    return _final(nump, denp2)


# speedup vs baseline: 35.1130x; 1.0073x over previous
import jax
import jax.numpy as jnp
from jax import lax
from jax.experimental import pallas as pl
from jax.experimental.pallas import tpu as pltpu
from jax.experimental.pallas import tpu_sc as plsc

N = 10000
E = 320000
F = 128
NC, NS, L = 2, 16, 16
NW = NC * NS
E_PER_W = E // NW
CHUNK = 80
N_CHUNKS = E_PER_W // CHUNK
NPAD = 10240
RPT = NPAD // NS

_MESH = plsc.VectorSubcoreMesh(
    core_axis_name="c", subcore_axis_name="s", num_cores=NC, num_subcores=NS)
_SC_PARAMS = pltpu.CompilerParams(needs_layout_passes=False)


def _dense1_body(x_ref, w_ref, a_ref, z_ref, st_ref):
    z = jnp.dot(x_ref[...], w_ref[...], preferred_element_type=jnp.float32)
    z_ref[...] = z
    aa = jnp.concatenate([a_ref[:F, :], a_ref[F:, :]], axis=1)
    st_ref[...] = jnp.dot(z, aa, preferred_element_type=jnp.float32)


def _dense1(x, W1, a1):
    return pl.pallas_call(
        _dense1_body,
        out_shape=[jax.ShapeDtypeStruct((N, F), jnp.float32),
                   jax.ShapeDtypeStruct((N, 2), jnp.float32)],
    )(x, W1, a1)


def _sc1_body(st_hbm, src_hbm, dst_hbm, z_hbm, zrow_hbm, zvec_hbm,
              acc_out, den_out, *sc):
    (acc_sh, den_sh, st_sh) = sc[0:3]
    SRC = sc[3:7]
    DST = sc[7:11]
    SIDX = sc[11:13]
    DIDX = sc[13:15]
    EX = sc[15:17]
    SV = sc[17:19]
    TV = sc[19:21]
    G = sc[21:23]
    S = sc[23:25]
    ISEM = sc[25:29]
    GSEM = sc[29:31]
    SSEM = sc[31:33]
    XSEM = sc[33:35]
    TSEM = sc[35:37]

    c = lax.axis_index("c")
    s = lax.axis_index("s")
    wid = c * NS + s
    row0 = s * RPT
    base = wid * E_PER_W
    pltpu.sync_copy(zrow_hbm, acc_sh.at[pl.ds(row0, RPT)])
    pltpu.sync_copy(zvec_hbm, den_sh.at[pl.ds(row0, RPT)])

    @pl.when(s == 0)
    def _():
        pltpu.sync_copy(st_hbm, st_sh)

    plsc.subcore_barrier()

    def start_idx(j, q):
        off = base + j * CHUNK
        pltpu.async_copy(src_hbm.at[pl.ds(off, CHUNK)], SRC[q], ISEM[q])
        pltpu.async_copy(dst_hbm.at[pl.ds(off, CHUNK)], DST[q], ISEM[q])

    def wait_idx(j, q):
        off = base + j * CHUNK
        pltpu.make_async_copy(src_hbm.at[pl.ds(off, CHUNK)], SRC[q],
                              ISEM[q]).wait()
        pltpu.make_async_copy(dst_hbm.at[pl.ds(off, CHUNK)], DST[q],
                              ISEM[q]).wait()

    def prep_st(b, q):
        for g in range(CHUNK // L):
            sl = pl.ds(g * L, L)
            SIDX[b][sl] = SRC[q][sl] * 2
            DIDX[b][sl] = DST[q][sl] * 2 + 1
        pltpu.async_copy(st_sh.at[SIDX[b]], SV[b], TSEM[b])
        pltpu.async_copy(st_sh.at[DIDX[b]], TV[b], TSEM[b])

    def wait_st(b):
        pltpu.make_async_copy(st_sh.at[SIDX[b]], SV[b], TSEM[b]).wait()
        pltpu.make_async_copy(st_sh.at[DIDX[b]], TV[b], TSEM[b]).wait()

    def start_gather(b, q):
        pltpu.async_copy(z_hbm.at[SRC[q]], G[b], GSEM[b])

    def wait_gather(b, q):
        pltpu.make_async_copy(z_hbm.at[SRC[q]], G[b], GSEM[b]).wait()

    def start_scatter(b, q):
        pltpu.async_copy(S[b], acc_sh.at[DST[q]], SSEM[b], add=True)
        pltpu.async_copy(EX[b], den_sh.at[DST[q]], XSEM[b], add=True)

    def wait_scatter(b, q):
        pltpu.make_async_copy(S[b], acc_sh.at[DST[q]], SSEM[b]).wait()
        pltpu.make_async_copy(EX[b], den_sh.at[DST[q]], XSEM[b]).wait()

    def compute_ex(b):
        for g in range(CHUNK // L):
            sl = pl.ds(g * L, L)
            e = SV[b][sl] + TV[b][sl]
            e = jnp.where(e >= 0.0, e, 0.2 * e)
            EX[b][sl] = jnp.exp(e)

    def scale(b):
        def scale_body(g, carry2):
            mv = EX[b][pl.ds(g * L, L)]

            def sub_body(u4, carry3):
                for k in range(4):
                    u = u4 * 4 + k
                    i = g * L + u
                    m = mv.at[jnp.full((L,), u, jnp.int32)].get(
                        mode="promise_in_bounds")
                    for v in range(F // L):
                        fs = pl.ds(v * L, L)
                        S[b][i, fs] = G[b][i, fs] * m
                return carry3

            lax.fori_loop(0, 4, sub_body, 0)
            return carry2

        lax.fori_loop(0, CHUNK // L, scale_body, 0)

    start_idx(0, 0)
    start_idx(1, 1)
    wait_idx(0, 0)
    prep_st(0, 0)
    start_gather(0, 0)
    wait_idx(1, 1)
    prep_st(1, 1)
    start_gather(1, 1)
    start_idx(2, 2)
    start_idx(3, 3)
    for j0 in (0, 1):
        b = q = j0
        wait_st(b)
        compute_ex(b)
        wait_gather(b, q)
        scale(b)
        start_scatter(b, q)
        wait_idx(j0 + 2, j0 + 2)
        prep_st(b, j0 + 2)
        start_gather(b, j0 + 2)

    def ring_body(i, carry):
        for b4 in range(4):
            j = 4 * i + 2 + b4
            b = b4 % 2
            q = (2 + b4) % 4
            qn = (q + 2) % 4
            wait_scatter(b, q)
            start_idx(j + 2, qn)
            wait_st(b)
            compute_ex(b)
            wait_gather(b, q)
            scale(b)
            start_scatter(b, q)
            wait_idx(j + 2, qn)
            prep_st(b, qn)
            start_gather(b, qn)
        return carry

    lax.fori_loop(0, 30, ring_body, 0)

    jt = jnp.int32(122)
    wait_scatter(0, 2)
    start_idx(jt + 2, 0)
    wait_st(0)
    compute_ex(0)
    wait_gather(0, 2)
    scale(0)
    start_scatter(0, 2)
    wait_idx(jt + 2, 0)
    prep_st(0, 0)
    start_gather(0, 0)
    wait_scatter(1, 3)
    wait_st(1)
    compute_ex(1)
    wait_gather(1, 3)
    scale(1)
    start_scatter(1, 3)
    wait_scatter(0, 0)
    wait_st(0)
    compute_ex(0)
    wait_gather(0, 0)
    scale(0)
    start_scatter(0, 0)
    wait_scatter(1, 3)
    wait_scatter(0, 0)

    plsc.subcore_barrier()
    pltpu.sync_copy(acc_sh.at[pl.ds(row0, RPT)],
                    acc_out.at[c, pl.ds(row0, RPT)])
    pltpu.sync_copy(den_sh.at[pl.ds(row0, RPT)],
                    den_out.at[c, pl.ds(row0, RPT)])


def _sc1(st_flat, src, dst, z1, zrow, zvec):
    return pl.kernel(
        _sc1_body,
        out_type=[jax.ShapeDtypeStruct((NC, NPAD, F), jnp.float32),
                  jax.ShapeDtypeStruct((NC, NPAD), jnp.float32)],
        mesh=_MESH,
        scratch_types=[
            pltpu.VMEM_SHARED((NPAD, F), jnp.float32),
            pltpu.VMEM_SHARED((NPAD,), jnp.float32),
            pltpu.VMEM_SHARED((2 * N,), jnp.float32),
        ] + [pltpu.VMEM((CHUNK,), jnp.int32)] * 12
          + [pltpu.VMEM((CHUNK,), jnp.float32)] * 6
          + [pltpu.VMEM((CHUNK, F), jnp.float32)] * 4
          + [pltpu.SemaphoreType.DMA] * 12,
        compiler_params=_SC_PARAMS,
    )(st_flat, src, dst, z1, zrow, zvec)


def _dense2_body(accp_ref, denp_ref, w2_ref, a2_ref, t3_ref):
    acc = accp_ref[0, :N, :] + accp_ref[1, :N, :]
    den = denp_ref[0, :N] + denp_ref[1, :N]
    rden = 1.0 / (den + 1e-9)
    h = jnp.maximum(acc * rden[:, None], 0.0)
    z2 = jnp.dot(h, w2_ref[...], preferred_element_type=jnp.float32)
    coef = jnp.concatenate(
        [jnp.ones((1, 1), jnp.float32), a2_ref[0:1, :], a2_ref[1:2, :]],
        axis=1)
    t3_ref[...] = z2 * coef


def _dense2(accp, denp, W2, a2):
    return pl.pallas_call(
        _dense2_body,
        out_shape=jax.ShapeDtypeStruct((N, 3), jnp.float32),
    )(accp, denp, W2, a2)


def _sc2_body(t3_hbm, src_hbm, dst_hbm, zvec_hbm,
              num_out, den_out, *sc):
    (num_sh, den_sh, t3_t) = sc[0:3]
    SRC = sc[3:7]
    DST = sc[7:11]
    EX = sc[11:13]
    NUM = sc[13:15]
    ISEM = sc[15:19]
    XSEM = sc[19:21]
    YSEM = sc[21:23]

    c = lax.axis_index("c")
    s = lax.axis_index("s")
    wid = c * NS + s
    row0 = s * RPT
    base = wid * E_PER_W
    pltpu.sync_copy(zvec_hbm, num_sh.at[pl.ds(row0, RPT)])
    pltpu.sync_copy(zvec_hbm, den_sh.at[pl.ds(row0, RPT)])
    pltpu.sync_copy(t3_hbm, t3_t)
    plsc.subcore_barrier()

    def start_idx(j, q):
        off = base + j * CHUNK
        pltpu.async_copy(src_hbm.at[pl.ds(off, CHUNK)], SRC[q], ISEM[q])
        pltpu.async_copy(dst_hbm.at[pl.ds(off, CHUNK)], DST[q], ISEM[q])

    def wait_idx(j, q):
        off = base + j * CHUNK
        pltpu.make_async_copy(src_hbm.at[pl.ds(off, CHUNK)], SRC[q],
                              ISEM[q]).wait()
        pltpu.make_async_copy(dst_hbm.at[pl.ds(off, CHUNK)], DST[q],
                              ISEM[q]).wait()

    def compute(b, q):
        for g in range(CHUNK // L):
            sl = pl.ds(g * L, L)
            si = SRC[q][sl]
            di = DST[q][sl]
            zs = plsc.load_gather(t3_t, [si * 3])
            ss = plsc.load_gather(t3_t, [si * 3 + 1])
            td = plsc.load_gather(t3_t, [di * 3 + 2])
            e = ss + td
            e = jnp.where(e >= 0.0, e, 0.2 * e)
            ex = jnp.exp(e)
            EX[b][sl] = ex
            NUM[b][sl] = ex * zs

    def start_scatter(b, q):
        pltpu.async_copy(EX[b], den_sh.at[DST[q]], XSEM[b], add=True)
        pltpu.async_copy(NUM[b], num_sh.at[DST[q]], YSEM[b], add=True)

    def wait_scatter(b, q):
        pltpu.make_async_copy(EX[b], den_sh.at[DST[q]], XSEM[b]).wait()
        pltpu.make_async_copy(NUM[b], num_sh.at[DST[q]], YSEM[b]).wait()

    start_idx(0, 0)
    start_idx(1, 1)
    start_idx(2, 2)
    start_idx(3, 3)
    for j0 in (0, 1):
        wait_idx(j0, j0)
        compute(j0, j0)
        start_scatter(j0, j0)

    def ring_body(i, carry):
        for b4 in range(4):
            j = 4 * i + 2 + b4
            b = b4 % 2
            q = (2 + b4) % 4
            qn = (q + 2) % 4
            wait_scatter(b, q)
            start_idx(j + 2, qn)
            wait_idx(j, q)
            compute(b, q)
            start_scatter(b, q)
        return carry

    lax.fori_loop(0, 30, ring_body, 0)

    jt = jnp.int32(122)
    wait_scatter(0, 2)
    start_idx(jt + 2, 0)
    wait_idx(jt, 2)
    compute(0, 2)
    start_scatter(0, 2)
    wait_scatter(1, 3)
    wait_idx(jt + 1, 3)
    compute(1, 3)
    start_scatter(1, 3)
    wait_scatter(0, 0)
    wait_idx(jt + 2, 0)
    compute(0, 0)
    start_scatter(0, 0)
    wait_scatter(1, 3)
    wait_scatter(0, 0)
    plsc.subcore_barrier()
    pltpu.sync_copy(num_sh.at[pl.ds(row0, RPT)],
                    num_out.at[c, pl.ds(row0, RPT)])
    pltpu.sync_copy(den_sh.at[pl.ds(row0, RPT)],
                    den_out.at[c, pl.ds(row0, RPT)])


def _sc2(t3_flat, src, dst, zvec):
    return pl.kernel(
        _sc2_body,
        out_type=[jax.ShapeDtypeStruct((NC, NPAD), jnp.float32),
                  jax.ShapeDtypeStruct((NC, NPAD), jnp.float32)],
        mesh=_MESH,
        scratch_types=[
            pltpu.VMEM_SHARED((NPAD,), jnp.float32),
            pltpu.VMEM_SHARED((NPAD,), jnp.float32),
            pltpu.VMEM((3 * N,), jnp.float32),
        ] + [pltpu.VMEM((CHUNK,), jnp.int32)] * 8
          + [pltpu.VMEM((CHUNK,), jnp.float32)] * 4
          + [pltpu.SemaphoreType.DMA] * 8,
        compiler_params=_SC_PARAMS,
    )(t3_flat, src, dst, zvec)


def _final_body(nump_ref, denp_ref, out_ref):
    num = nump_ref[0, :N] + nump_ref[1, :N]
    den = denp_ref[0, :N] + denp_ref[1, :N]
    r = num / (den + 1e-9)
    out_ref[...] = jax.nn.sigmoid(r)[:, None]


def _final(nump, denp):
    return pl.pallas_call(
        _final_body,
        out_shape=jax.ShapeDtypeStruct((N, 1), jnp.float32),
    )(nump, denp)


def kernel(edge_index, x, W1, a1, W2, a2):
    ei = edge_index.astype(jnp.int32)
    src = ei[0]
    dst = ei[1]
    zrow = jnp.zeros((RPT, F), jnp.float32)
    zvec = jnp.zeros((RPT,), jnp.float32)

    z1, st1 = _dense1(x, W1, a1)
    accp, denp = _sc1(st1.reshape(2 * N), src, dst, z1, zrow, zvec)
    t3 = _dense2(accp, denp, W2, a2)
    nump, denp2 = _sc2(t3.reshape(3 * N), src, dst, zvec)
    return _final(nump, denp2)


# speedup vs baseline: 69.3712x; 1.9757x over previous
import jax
import jax.numpy as jnp
from jax import lax
from jax.experimental import pallas as pl
from jax.experimental.pallas import tpu as pltpu
from jax.experimental.pallas import tpu_sc as plsc

N = 10000
E = 320000
F = 128
NC, NS, L = 2, 16, 16
NW = NC * NS
E_PER_W = E // NW
CHUNK = 80
N_CHUNKS = E_PER_W // CHUNK
NPAD = 10240
RPT = NPAD // NS

_MESH = plsc.VectorSubcoreMesh(
    core_axis_name="c", subcore_axis_name="s", num_cores=NC, num_subcores=NS)
_SC_PARAMS = pltpu.CompilerParams(needs_layout_passes=False)


def _dense1_body(x_ref, w_ref, a_ref, z_ref, st_ref):
    z = jnp.dot(x_ref[...], w_ref[...], preferred_element_type=jnp.float32)
    z_ref[...] = z
    aa = jnp.concatenate([a_ref[:F, :], a_ref[F:, :]], axis=1)
    st_ref[...] = jnp.dot(z, aa, preferred_element_type=jnp.float32)


def _dense1(x, W1, a1):
    return pl.pallas_call(
        _dense1_body,
        out_shape=[jax.ShapeDtypeStruct((N, F), jnp.float32),
                   jax.ShapeDtypeStruct((N, 2), jnp.float32)],
    )(x, W1, a1)


def _sc1_body(st_hbm, src_hbm, dst_hbm, z_hbm, zrow_hbm, zvec_hbm,
              acc_out, den_out, *sc):
    (acc_sh, den_sh, st_sh) = sc[0:3]
    SRC = sc[3:7]
    DST = sc[7:11]
    SIDX = sc[11:13]
    DIDX = sc[13:15]
    EX = sc[15:17]
    SV = sc[17:19]
    TV = sc[19:21]
    G = sc[21:23]
    S = sc[23:25]
    ISEM = sc[25:29]
    GSEM = sc[29:31]
    SSEM = sc[31:33]
    XSEM = sc[33:35]
    TSEM = sc[35:37]

    c = lax.axis_index("c")
    s = lax.axis_index("s")
    wid = c * NS + s
    row0 = s * RPT
    base = wid * E_PER_W
    pltpu.sync_copy(zrow_hbm, acc_sh.at[pl.ds(row0, RPT)])
    pltpu.sync_copy(zvec_hbm, den_sh.at[pl.ds(row0, RPT)])

    @pl.when(s == 0)
    def _():
        pltpu.sync_copy(st_hbm, st_sh)

    plsc.subcore_barrier()

    def start_idx(j, q):
        off = base + j * CHUNK
        pltpu.async_copy(src_hbm.at[pl.ds(off, CHUNK)], SRC[q], ISEM[q])
        pltpu.async_copy(dst_hbm.at[pl.ds(off, CHUNK)], DST[q], ISEM[q])

    def wait_idx(j, q):
        off = base + j * CHUNK
        pltpu.make_async_copy(src_hbm.at[pl.ds(off, CHUNK)], SRC[q],
                              ISEM[q]).wait()
        pltpu.make_async_copy(dst_hbm.at[pl.ds(off, CHUNK)], DST[q],
                              ISEM[q]).wait()

    def prep_st(b, q):
        for g in range(CHUNK // L):
            sl = pl.ds(g * L, L)
            SIDX[b][sl] = SRC[q][sl] * 2
            DIDX[b][sl] = DST[q][sl] * 2 + 1
        pltpu.async_copy(st_sh.at[SIDX[b]], SV[b], TSEM[b])
        pltpu.async_copy(st_sh.at[DIDX[b]], TV[b], TSEM[b])

    def wait_st(b):
        pltpu.make_async_copy(st_sh.at[SIDX[b]], SV[b], TSEM[b]).wait()
        pltpu.make_async_copy(st_sh.at[DIDX[b]], TV[b], TSEM[b]).wait()

    def start_gather(b, q):
        pltpu.async_copy(z_hbm.at[SRC[q]], G[b], GSEM[b])

    def wait_gather(b, q):
        pltpu.make_async_copy(z_hbm.at[SRC[q]], G[b], GSEM[b]).wait()

    def start_scatter(b, q):
        pltpu.async_copy(S[b], acc_sh.at[DST[q]], SSEM[b], add=True)
        pltpu.async_copy(EX[b], den_sh.at[DST[q]], XSEM[b], add=True)

    def wait_scatter(b, q):
        pltpu.make_async_copy(S[b], acc_sh.at[DST[q]], SSEM[b]).wait()
        pltpu.make_async_copy(EX[b], den_sh.at[DST[q]], XSEM[b]).wait()

    def compute_ex(b):
        for g in range(CHUNK // L):
            sl = pl.ds(g * L, L)
            e = SV[b][sl] + TV[b][sl]
            e = jnp.where(e >= 0.0, e, 0.2 * e)
            EX[b][sl] = jnp.exp(e)

    def scale(b):
        @plsc.parallel_loop(0, CHUNK, step=L)
        def _(e0):
            mv = EX[b][pl.ds(e0, L)]
            for u in range(L):
                m = mv.at[jnp.full((L,), u, jnp.int32)].get(
                    mode="promise_in_bounds")
                for v in range(F // L):
                    fs = pl.ds(v * L, L)
                    S[b][e0 + u, fs] = G[b][e0 + u, fs] * m

    start_idx(0, 0)
    start_idx(1, 1)
    wait_idx(0, 0)
    prep_st(0, 0)
    start_gather(0, 0)
    wait_idx(1, 1)
    prep_st(1, 1)
    start_gather(1, 1)
    start_idx(2, 2)
    start_idx(3, 3)
    for j0 in (0, 1):
        b = q = j0
        wait_st(b)
        compute_ex(b)
        wait_gather(b, q)
        scale(b)
        start_scatter(b, q)
        wait_idx(j0 + 2, j0 + 2)
        prep_st(b, j0 + 2)
        start_gather(b, j0 + 2)

    def ring_body(i, carry):
        for b4 in range(4):
            j = 4 * i + 2 + b4
            b = b4 % 2
            q = (2 + b4) % 4
            qn = (q + 2) % 4
            wait_scatter(b, q)
            start_idx(j + 2, qn)
            wait_st(b)
            compute_ex(b)
            wait_gather(b, q)
            scale(b)
            start_scatter(b, q)
            wait_idx(j + 2, qn)
            prep_st(b, qn)
            start_gather(b, qn)
        return carry

    lax.fori_loop(0, 30, ring_body, 0)

    jt = jnp.int32(122)
    wait_scatter(0, 2)
    start_idx(jt + 2, 0)
    wait_st(0)
    compute_ex(0)
    wait_gather(0, 2)
    scale(0)
    start_scatter(0, 2)
    wait_idx(jt + 2, 0)
    prep_st(0, 0)
    start_gather(0, 0)
    wait_scatter(1, 3)
    wait_st(1)
    compute_ex(1)
    wait_gather(1, 3)
    scale(1)
    start_scatter(1, 3)
    wait_scatter(0, 0)
    wait_st(0)
    compute_ex(0)
    wait_gather(0, 0)
    scale(0)
    start_scatter(0, 0)
    wait_scatter(1, 3)
    wait_scatter(0, 0)

    plsc.subcore_barrier()
    pltpu.sync_copy(acc_sh.at[pl.ds(row0, RPT)],
                    acc_out.at[c, pl.ds(row0, RPT)])
    pltpu.sync_copy(den_sh.at[pl.ds(row0, RPT)],
                    den_out.at[c, pl.ds(row0, RPT)])


def _sc1(st_flat, src, dst, z1, zrow, zvec):
    return pl.kernel(
        _sc1_body,
        out_type=[jax.ShapeDtypeStruct((NC, NPAD, F), jnp.float32),
                  jax.ShapeDtypeStruct((NC, NPAD), jnp.float32)],
        mesh=_MESH,
        scratch_types=[
            pltpu.VMEM_SHARED((NPAD, F), jnp.float32),
            pltpu.VMEM_SHARED((NPAD,), jnp.float32),
            pltpu.VMEM_SHARED((2 * N,), jnp.float32),
        ] + [pltpu.VMEM((CHUNK,), jnp.int32)] * 12
          + [pltpu.VMEM((CHUNK,), jnp.float32)] * 6
          + [pltpu.VMEM((CHUNK, F), jnp.float32)] * 4
          + [pltpu.SemaphoreType.DMA] * 12,
        compiler_params=_SC_PARAMS,
    )(st_flat, src, dst, z1, zrow, zvec)


def _dense2_body(accp_ref, denp_ref, w2_ref, a2_ref, t3_ref):
    acc = accp_ref[0, :N, :] + accp_ref[1, :N, :]
    den = denp_ref[0, :N] + denp_ref[1, :N]
    rden = 1.0 / (den + 1e-9)
    h = jnp.maximum(acc * rden[:, None], 0.0)
    z2 = jnp.dot(h, w2_ref[...], preferred_element_type=jnp.float32)
    coef = jnp.concatenate(
        [jnp.ones((1, 1), jnp.float32), a2_ref[0:1, :], a2_ref[1:2, :]],
        axis=1)
    t3_ref[...] = z2 * coef


def _dense2(accp, denp, W2, a2):
    return pl.pallas_call(
        _dense2_body,
        out_shape=jax.ShapeDtypeStruct((N, 3), jnp.float32),
    )(accp, denp, W2, a2)


def _sc2_body(t3_hbm, src_hbm, dst_hbm, zvec_hbm,
              num_out, den_out, *sc):
    (num_sh, den_sh, t3_t) = sc[0:3]
    SRC = sc[3:7]
    DST = sc[7:11]
    EX = sc[11:13]
    NUM = sc[13:15]
    ISEM = sc[15:19]
    XSEM = sc[19:21]
    YSEM = sc[21:23]

    c = lax.axis_index("c")
    s = lax.axis_index("s")
    wid = c * NS + s
    row0 = s * RPT
    base = wid * E_PER_W
    pltpu.sync_copy(zvec_hbm, num_sh.at[pl.ds(row0, RPT)])
    pltpu.sync_copy(zvec_hbm, den_sh.at[pl.ds(row0, RPT)])
    pltpu.sync_copy(t3_hbm, t3_t)
    plsc.subcore_barrier()

    def start_idx(j, q):
        off = base + j * CHUNK
        pltpu.async_copy(src_hbm.at[pl.ds(off, CHUNK)], SRC[q], ISEM[q])
        pltpu.async_copy(dst_hbm.at[pl.ds(off, CHUNK)], DST[q], ISEM[q])

    def wait_idx(j, q):
        off = base + j * CHUNK
        pltpu.make_async_copy(src_hbm.at[pl.ds(off, CHUNK)], SRC[q],
                              ISEM[q]).wait()
        pltpu.make_async_copy(dst_hbm.at[pl.ds(off, CHUNK)], DST[q],
                              ISEM[q]).wait()

    def compute(b, q):
        for g in range(CHUNK // L):
            sl = pl.ds(g * L, L)
            si = SRC[q][sl]
            di = DST[q][sl]
            zs = plsc.load_gather(t3_t, [si * 3])
            ss = plsc.load_gather(t3_t, [si * 3 + 1])
            td = plsc.load_gather(t3_t, [di * 3 + 2])
            e = ss + td
            e = jnp.where(e >= 0.0, e, 0.2 * e)
            ex = jnp.exp(e)
            EX[b][sl] = ex
            NUM[b][sl] = ex * zs

    def start_scatter(b, q):
        pltpu.async_copy(EX[b], den_sh.at[DST[q]], XSEM[b], add=True)
        pltpu.async_copy(NUM[b], num_sh.at[DST[q]], YSEM[b], add=True)

    def wait_scatter(b, q):
        pltpu.make_async_copy(EX[b], den_sh.at[DST[q]], XSEM[b]).wait()
        pltpu.make_async_copy(NUM[b], num_sh.at[DST[q]], YSEM[b]).wait()

    start_idx(0, 0)
    start_idx(1, 1)
    start_idx(2, 2)
    start_idx(3, 3)
    for j0 in (0, 1):
        wait_idx(j0, j0)
        compute(j0, j0)
        start_scatter(j0, j0)

    def ring_body(i, carry):
        for b4 in range(4):
            j = 4 * i + 2 + b4
            b = b4 % 2
            q = (2 + b4) % 4
            qn = (q + 2) % 4
            wait_scatter(b, q)
            start_idx(j + 2, qn)
            wait_idx(j, q)
            compute(b, q)
            start_scatter(b, q)
        return carry

    lax.fori_loop(0, 30, ring_body, 0)

    jt = jnp.int32(122)
    wait_scatter(0, 2)
    start_idx(jt + 2, 0)
    wait_idx(jt, 2)
    compute(0, 2)
    start_scatter(0, 2)
    wait_scatter(1, 3)
    wait_idx(jt + 1, 3)
    compute(1, 3)
    start_scatter(1, 3)
    wait_scatter(0, 0)
    wait_idx(jt + 2, 0)
    compute(0, 0)
    start_scatter(0, 0)
    wait_scatter(1, 3)
    wait_scatter(0, 0)
    plsc.subcore_barrier()
    pltpu.sync_copy(num_sh.at[pl.ds(row0, RPT)],
                    num_out.at[c, pl.ds(row0, RPT)])
    pltpu.sync_copy(den_sh.at[pl.ds(row0, RPT)],
                    den_out.at[c, pl.ds(row0, RPT)])


def _sc2(t3_flat, src, dst, zvec):
    return pl.kernel(
        _sc2_body,
        out_type=[jax.ShapeDtypeStruct((NC, NPAD), jnp.float32),
                  jax.ShapeDtypeStruct((NC, NPAD), jnp.float32)],
        mesh=_MESH,
        scratch_types=[
            pltpu.VMEM_SHARED((NPAD,), jnp.float32),
            pltpu.VMEM_SHARED((NPAD,), jnp.float32),
            pltpu.VMEM((3 * N,), jnp.float32),
        ] + [pltpu.VMEM((CHUNK,), jnp.int32)] * 8
          + [pltpu.VMEM((CHUNK,), jnp.float32)] * 4
          + [pltpu.SemaphoreType.DMA] * 8,
        compiler_params=_SC_PARAMS,
    )(t3_flat, src, dst, zvec)


def _final_body(nump_ref, denp_ref, out_ref):
    num = nump_ref[0, :N] + nump_ref[1, :N]
    den = denp_ref[0, :N] + denp_ref[1, :N]
    r = num / (den + 1e-9)
    out_ref[...] = jax.nn.sigmoid(r)[:, None]


def _final(nump, denp):
    return pl.pallas_call(
        _final_body,
        out_shape=jax.ShapeDtypeStruct((N, 1), jnp.float32),
    )(nump, denp)


def kernel(edge_index, x, W1, a1, W2, a2):
    ei = edge_index.astype(jnp.int32)
    src = ei[0]
    dst = ei[1]
    zrow = jnp.zeros((RPT, F), jnp.float32)
    zvec = jnp.zeros((RPT,), jnp.float32)

    z1, st1 = _dense1(x, W1, a1)
    accp, denp = _sc1(st1.reshape(2 * N), src, dst, z1, zrow, zvec)
    t3 = _dense2(accp, denp, W2, a2)
    nump, denp2 = _sc2(t3.reshape(3 * N), src, dst, zvec)
    return _final(nump, denp2)
